# baseline (speedup 1.0000x reference)
import functools
import jax
import jax.numpy as jnp
from jax import lax
from jax.experimental import pallas as pl
from jax.experimental.pallas import tpu as pltpu
from jax.experimental.pallas import tpu_sc as plsc

N = 2048
D = 768
E = 8
DFF = 3072
K = 2
NK = N * K
BLK = 128
M = NK + E * BLK
NB = M // BLK
FB = 512
NFB = DFF // FB
CH = 16
NCH = N // CH
V_MAX = 3.0
FEAT_CLAMP = 10.0
NEG_BIG = -1e30
NW = 32


def _router_body(x_ref, wr_ref, br_ref, i12_ref, g12_ref):
    logits = jnp.dot(x_ref[...], wr_ref[...],
                     preferred_element_type=jnp.float32) + br_ref[...]
    idx = jax.lax.broadcasted_iota(jnp.int32, (N, E), 1)
    m1 = jnp.max(logits, axis=1, keepdims=True)
    i1 = jnp.min(jnp.where(logits == m1, idx, E), axis=1, keepdims=True)
    l2 = jnp.where(idx == i1, NEG_BIG, logits)
    m2 = jnp.max(l2, axis=1, keepdims=True)
    i2 = jnp.min(jnp.where(l2 == m2, idx, E), axis=1, keepdims=True)
    e2 = jnp.exp(m2 - m1)
    denom = 1.0 + e2
    i12_ref[...] = jnp.concatenate([i1, i2], axis=1)
    g12_ref[...] = jnp.concatenate([1.0 / denom, e2 / denom], axis=1)


def _router(x2, Wr, br):
    return pl.pallas_call(
        _router_body,
        grid=(1,),
        in_specs=[
            pl.BlockSpec((N, D), lambda i: (0, 0)),
            pl.BlockSpec((D, E), lambda i: (0, 0)),
            pl.BlockSpec((1, E), lambda i: (0, 0)),
        ],
        out_specs=[
            pl.BlockSpec((N, K), lambda i: (0, 0)),
            pl.BlockSpec((N, K), lambda i: (0, 0)),
        ],
        out_shape=[
            jax.ShapeDtypeStruct((N, K), jnp.int32),
            jax.ShapeDtypeStruct((N, K), jnp.float32),
        ],
    )(x2, Wr, br.reshape(1, E))


def _sc_gather(table, idx):
    rows, _ = table.shape
    b = idx.shape[0]
    rpw = b // NW
    mesh = plsc.VectorSubcoreMesh(core_axis_name="c", subcore_axis_name="s")

    @functools.partial(
        pl.kernel, mesh=mesh,
        out_type=jax.ShapeDtypeStruct((b, D), jnp.float32),
        scratch_types=[
            pltpu.VMEM((rpw,), jnp.int32),
            pltpu.VMEM((rpw, D), jnp.float32),
            pltpu.SemaphoreType.DMA,
        ],
    )
    def k(table_hbm, idx_hbm, out_hbm, idx_v, rows_v, sem):
        wid = lax.axis_index("s") * 2 + lax.axis_index("c")
        base = wid * rpw
        pltpu.sync_copy(idx_hbm.at[pl.ds(base, rpw)], idx_v)
        pltpu.async_copy(table_hbm.at[idx_v], rows_v, sem).wait()
        pltpu.sync_copy(rows_v, out_hbm.at[pl.ds(base, rpw)])

    return k(table, idx)


def _gffn_body(be_ref, xg_ref, gate_ref, w1_ref, b1_ref, w2_ref, b2_ref,
               ys_ref):
    f = pl.program_id(0)
    b = pl.program_id(1)
    rows = pl.ds(b * BLK, BLK)
    g = gate_ref[rows, :]
    h = jnp.dot(xg_ref[rows, :], w1_ref[0],
                preferred_element_type=jnp.float32)
    h = jnp.maximum(h + b1_ref[0], 0.0) * g
    part = jnp.dot(h, w2_ref[0], preferred_element_type=jnp.float32)

    @pl.when(f == 0)
    def _init():
        ys_ref[rows, :] = g * b2_ref[0] + part

    @pl.when(f != 0)
    def _acc():
        ys_ref[rows, :] += part


def _grouped_ffn(be, xg, gates_m, W1, b1, W2, b2):
    grid_spec = pltpu.PrefetchScalarGridSpec(
        num_scalar_prefetch=1,
        grid=(NFB, NB),
        in_specs=[
            pl.BlockSpec((M, D), lambda f, b, be: (0, 0)),
            pl.BlockSpec((M, 1), lambda f, b, be: (0, 0)),
            pl.BlockSpec((1, D, FB), lambda f, b, be: (be[b], 0, f)),
            pl.BlockSpec((1, 1, FB), lambda f, b, be: (be[b], 0, f)),
            pl.BlockSpec((1, FB, D), lambda f, b, be: (be[b], f, 0)),
            pl.BlockSpec((1, 1, D), lambda f, b, be: (be[b], 0, 0)),
        ],
        out_specs=pl.BlockSpec((M, D), lambda f, b, be: (0, 0)),
    )
    return pl.pallas_call(
        _gffn_body,
        grid_spec=grid_spec,
        out_shape=jax.ShapeDtypeStruct((M, D), jnp.float32),
    )(be, xg, gates_m, W1, b1.reshape(E, 1, DFF), W2, b2.reshape(E, 1, D))


def _sumv_body(yg_ref, wv_ref, bv_ref, ffn_ref, v_ref):
    ffn = yg_ref[0] + yg_ref[1]
    ffn_ref[...] = ffn
    v = jnp.dot(ffn, wv_ref[...], preferred_element_type=jnp.float32)
    v_ref[...] = jnp.clip(v + bv_ref[0, 0], -V_MAX, V_MAX)


def _sumv(yg2, Wv, bv):
    return pl.pallas_call(
        _sumv_body,
        grid=(1,),
        in_specs=[
            pl.BlockSpec((2, N, D), lambda i: (0, 0, 0)),
            pl.BlockSpec((D, 1), lambda i: (0, 0)),
            pl.BlockSpec(memory_space=pltpu.SMEM),
        ],
        out_specs=[
            pl.BlockSpec((N, D), lambda i: (0, 0)),
            pl.BlockSpec((N, 1), lambda i: (0, 0)),
        ],
        out_shape=[
            jax.ShapeDtypeStruct((N, D), jnp.float32),
            jax.ShapeDtypeStruct((N, 1), jnp.float32),
        ],
    )(yg2, Wv, bv.reshape(1, 1))


def _directional_scan(dr, mat_ref, start_ref, out_re_ref, out_im_ref):
    one = jnp.ones((NCH, 1), jnp.float32)
    zero = jnp.zeros((NCH, 1), jnp.float32)
    m00r, m00i = one, zero
    m01r, m01i = zero, zero
    m10r, m10i = zero, zero
    m11r, m11i = one, zero
    for j in range(CH):
        dj = dr[:, j:j + 1]
        n10r = dj * m10r - m10i - m00r
        n10i = dj * m10i + m10r - m00i
        n11r = dj * m11r - m11i - m01r
        n11i = dj * m11i + m11r - m01i
        m00r, m00i = m10r, m10i
        m01r, m01i = m11r, m11i
        m10r, m10i = n10r, n10i
        m11r, m11i = n11r, n11i
    mat_ref[:, 0:1] = m00r
    mat_ref[:, 1:2] = m00i
    mat_ref[:, 2:3] = m01r
    mat_ref[:, 3:4] = m01i
    mat_ref[:, 4:5] = m10r
    mat_ref[:, 5:6] = m10i
    mat_ref[:, 6:7] = m11r
    mat_ref[:, 7:8] = m11i

    def boundary(c, carry):
        are, aim = carry
        start_ref[pl.ds(c, 1), :] = jnp.concatenate([are, aim], axis=1)
        row = mat_ref[pl.ds(c, 1), :]
        numr = row[:, 0:1] * are - row[:, 1:2] * aim + row[:, 2:3]
        numi = row[:, 0:1] * aim + row[:, 1:2] * are + row[:, 3:4]
        denr = row[:, 4:5] * are - row[:, 5:6] * aim + row[:, 6:7]
        deni = row[:, 4:5] * aim + row[:, 5:6] * are + row[:, 7:8]
        nrm = denr * denr + deni * deni
        return ((numr * denr + numi * deni) / nrm,
                (numi * denr - numr * deni) / nrm)

    z11 = jnp.zeros((1, 1), jnp.float32)
    jax.lax.fori_loop(0, NCH, boundary, (z11, z11))

    are = start_ref[:, 0:1]
    aim = start_ref[:, 1:2]
    for j in range(CH):
        out_re_ref[:, j:j + 1] = are
        out_im_ref[:, j:j + 1] = aim
        x = dr[:, j:j + 1] - are
        y = 1.0 - aim
        nrm = x * x + y * y
        are = x / nrm
        aim = -y / nrm


def _scan_body(v_ref, vrev_ref, are_ref, aim_ref, bre_ref, bim_ref,
               mat_ref, start_ref):
    _directional_scan(2.0 - v_ref[...], mat_ref, start_ref, are_ref, aim_ref)
    _directional_scan(2.0 - vrev_ref[...], mat_ref, start_ref,
                      bre_ref, bim_ref)


def _bk_scan(v16, v16rev):
    return pl.pallas_call(
        _scan_body,
        grid=(1,),
        in_specs=[
            pl.BlockSpec((NCH, CH), lambda i: (0, 0)),
            pl.BlockSpec((NCH, CH), lambda i: (0, 0)),
        ],
        out_specs=[pl.BlockSpec((NCH, CH), lambda i: (0, 0))] * 4,
        out_shape=[jax.ShapeDtypeStruct((NCH, CH), jnp.float32)] * 4,
        scratch_shapes=[
            pltpu.VMEM((NCH, 8), jnp.float32),
            pltpu.VMEM((NCH, 2), jnp.float32),
        ],
    )(v16, v16rev)


def _combine_body(ffn_ref, v_ref, are_ref, aim_ref, rre_ref, rim_ref,
                  wo_ref, bo_ref, sc_ref, out_ref):
    x = (2.0 - v_ref[...]) - are_ref[...] - rre_ref[...]
    y = 1.0 - aim_ref[...] - rim_ref[...]
    nrm = x * x + y * y
    gre = jnp.clip(x / nrm, -FEAT_CLAMP, FEAT_CLAMP)
    gim = jnp.clip(-y / nrm, -FEAT_CLAMP, FEAT_CLAMP)
    spec = gre * wo_ref[0:1, :] + gim * wo_ref[1:2, :] + bo_ref[...]
    out_ref[...] = ffn_ref[...] + sc_ref[0, 0] * spec


def _combine(ffn, v2, are, aim, rre, rim, Wo, bo, bk_scale):
    return pl.pallas_call(
        _combine_body,
        grid=(1,),
        in_specs=[
            pl.BlockSpec((N, D), lambda i: (0, 0)),
            pl.BlockSpec((N, 1), lambda i: (0, 0)),
            pl.BlockSpec((N, 1), lambda i: (0, 0)),
            pl.BlockSpec((N, 1), lambda i: (0, 0)),
            pl.BlockSpec((N, 1), lambda i: (0, 0)),
            pl.BlockSpec((N, 1), lambda i: (0, 0)),
            pl.BlockSpec((2, D), lambda i: (0, 0)),
            pl.BlockSpec((1, D), lambda i: (0, 0)),
            pl.BlockSpec(memory_space=pltpu.SMEM),
        ],
        out_specs=pl.BlockSpec((N, D), lambda i: (0, 0)),
        out_shape=jax.ShapeDtypeStruct((N, D), jnp.float32),
    )(ffn, v2, are, aim, rre, rim, Wo, bo.reshape(1, D),
      bk_scale.reshape(1, 1))


def _dispatch_meta(i12, g12):
    eflat = i12.reshape(NK)
    gflat = g12.reshape(NK)
    oh = (eflat[:, None] == jnp.arange(E, dtype=jnp.int32)[None, :])
    oh = oh.astype(jnp.int32)
    csum = jnp.cumsum(oh, axis=0)
    rank = jnp.sum(oh * (csum - 1), axis=1)
    counts = csum[-1]
    padded = ((counts + BLK - 1) // BLK) * BLK
    pstart = jnp.concatenate(
        [jnp.zeros((1,), jnp.int32), jnp.cumsum(padded)])[:E]
    pos = pstart[eflat] + rank
    tok = jnp.arange(NK, dtype=jnp.int32) // K
    tok_ids = jnp.zeros((M,), jnp.int32).at[pos].set(tok)
    gates_m = jnp.zeros((M,), jnp.float32).at[pos].set(gflat)
    bstart = pstart // BLK
    be = jnp.sum(jnp.arange(NB, dtype=jnp.int32)[:, None] >= bstart[None, :],
                 axis=1).astype(jnp.int32) - 1
    pos_t = pos.reshape(N, K).T.reshape(NK)
    return tok_ids, gates_m, be, pos_t


def kernel(x, Wr, br, W1, b1, W2, b2, Wv, bv, Wo, bo, bk_scale):
    x2 = x.reshape(N, D)
    i12, g12 = _router(x2, Wr, br)
    tok_ids, gates_m, be, pos_t = _dispatch_meta(i12, g12)
    xg = _sc_gather(x2, tok_ids)
    ys = _grouped_ffn(be, xg, gates_m.reshape(M, 1), W1, b1, W2, b2)
    yg = _sc_gather(ys, pos_t)
    ffn, v2 = _sumv(yg.reshape(2, N, D), Wv, bv)
    v16 = v2.reshape(NCH, CH)
    v16rev = v16[::-1, ::-1]
    are, aim, bre, bim = _bk_scan(v16, v16rev)
    rre = bre[::-1, ::-1].reshape(N, 1)
    rim = bim[::-1, ::-1].reshape(N, 1)
    out = _combine(ffn, v2, are.reshape(N, 1), aim.reshape(N, 1),
                   rre, rim, Wo, bo, bk_scale)
    return out.reshape(x.shape)


# speedup vs baseline: 1.0042x; 1.0042x over previous
import functools
import jax
import jax.numpy as jnp
from jax import lax
from jax.experimental import pallas as pl
from jax.experimental.pallas import tpu as pltpu
from jax.experimental.pallas import tpu_sc as plsc

N = 2048
D = 768
E = 8
DFF = 3072
K = 2
NK = N * K
BLK = 128
M = NK + E * BLK
NB = M // BLK
FB = 512
NFB = DFF // FB
CH = 16
NCH = N // CH
V_MAX = 3.0
FEAT_CLAMP = 10.0
NEG_BIG = -1e30
NW = 32


def _router_body(x_ref, wr_ref, br_ref, i12_ref, g12_ref):
    logits = jnp.dot(x_ref[...], wr_ref[...],
                     preferred_element_type=jnp.float32) + br_ref[...]
    idx = jax.lax.broadcasted_iota(jnp.int32, (N, E), 1)
    m1 = jnp.max(logits, axis=1, keepdims=True)
    i1 = jnp.min(jnp.where(logits == m1, idx, E), axis=1, keepdims=True)
    l2 = jnp.where(idx == i1, NEG_BIG, logits)
    m2 = jnp.max(l2, axis=1, keepdims=True)
    i2 = jnp.min(jnp.where(l2 == m2, idx, E), axis=1, keepdims=True)
    e2 = jnp.exp(m2 - m1)
    denom = 1.0 + e2
    i12_ref[...] = jnp.concatenate([i1, i2], axis=1)
    g12_ref[...] = jnp.concatenate([1.0 / denom, e2 / denom], axis=1)


def _router(x2, Wr, br):
    return pl.pallas_call(
        _router_body,
        grid=(1,),
        in_specs=[
            pl.BlockSpec((N, D), lambda i: (0, 0)),
            pl.BlockSpec((D, E), lambda i: (0, 0)),
            pl.BlockSpec((1, E), lambda i: (0, 0)),
        ],
        out_specs=[
            pl.BlockSpec((N, K), lambda i: (0, 0)),
            pl.BlockSpec((N, K), lambda i: (0, 0)),
        ],
        out_shape=[
            jax.ShapeDtypeStruct((N, K), jnp.int32),
            jax.ShapeDtypeStruct((N, K), jnp.float32),
        ],
    )(x2, Wr, br.reshape(1, E))


def _sc_gather(table, idx):
    rows, _ = table.shape
    b = idx.shape[0]
    rpw = b // NW
    mesh = plsc.VectorSubcoreMesh(core_axis_name="c", subcore_axis_name="s")

    @functools.partial(
        pl.kernel, mesh=mesh,
        out_type=jax.ShapeDtypeStruct((b, D), jnp.float32),
        scratch_types=[
            pltpu.VMEM((rpw,), jnp.int32),
            pltpu.VMEM((rpw, D), jnp.float32),
            pltpu.SemaphoreType.DMA,
        ],
    )
    def k(table_hbm, idx_hbm, out_hbm, idx_v, rows_v, sem):
        wid = lax.axis_index("s") * 2 + lax.axis_index("c")
        base = wid * rpw
        pltpu.sync_copy(idx_hbm.at[pl.ds(base, rpw)], idx_v)
        pltpu.async_copy(table_hbm.at[idx_v], rows_v, sem).wait()
        pltpu.sync_copy(rows_v, out_hbm.at[pl.ds(base, rpw)])

    return k(table, idx)


def _gffn_body(be_ref, xg_ref, gate_ref, w1_ref, b1_ref, w2_ref, b2_ref,
               ys_ref):
    f = pl.program_id(0)
    b = pl.program_id(1)
    rows = pl.ds(b * BLK, BLK)
    g = gate_ref[rows, :]
    h = jnp.dot(xg_ref[rows, :], w1_ref[0],
                preferred_element_type=jnp.float32)
    h = jnp.maximum(h + b1_ref[0], 0.0) * g
    part = jnp.dot(h, w2_ref[0], preferred_element_type=jnp.float32)

    @pl.when(f == 0)
    def _init():
        ys_ref[rows, :] = g * b2_ref[0] + part

    @pl.when(f != 0)
    def _acc():
        ys_ref[rows, :] += part


def _grouped_ffn(be, xg, gates_m, W1, b1, W2, b2):
    grid_spec = pltpu.PrefetchScalarGridSpec(
        num_scalar_prefetch=1,
        grid=(NFB, NB),
        in_specs=[
            pl.BlockSpec((M, D), lambda f, b, be: (0, 0)),
            pl.BlockSpec((M, 1), lambda f, b, be: (0, 0)),
            pl.BlockSpec((1, D, FB), lambda f, b, be: (be[b], 0, f)),
            pl.BlockSpec((1, 1, FB), lambda f, b, be: (be[b], 0, f)),
            pl.BlockSpec((1, FB, D), lambda f, b, be: (be[b], f, 0)),
            pl.BlockSpec((1, 1, D), lambda f, b, be: (be[b], 0, 0)),
        ],
        out_specs=pl.BlockSpec((M, D), lambda f, b, be: (0, 0)),
    )
    return pl.pallas_call(
        _gffn_body,
        grid_spec=grid_spec,
        out_shape=jax.ShapeDtypeStruct((M, D), jnp.float32),
    )(be, xg, gates_m, W1, b1.reshape(E, 1, DFF), W2, b2.reshape(E, 1, D))


def _sumv_body(yg_ref, wv_ref, bv_ref, ffn_ref, v_ref):
    ffn = yg_ref[0] + yg_ref[1]
    ffn_ref[...] = ffn
    v = jnp.dot(ffn, wv_ref[...], preferred_element_type=jnp.float32)
    v_ref[...] = jnp.clip(v + bv_ref[0, 0], -V_MAX, V_MAX)


def _sumv(yg2, Wv, bv):
    return pl.pallas_call(
        _sumv_body,
        grid=(1,),
        in_specs=[
            pl.BlockSpec((2, N, D), lambda i: (0, 0, 0)),
            pl.BlockSpec((D, 1), lambda i: (0, 0)),
            pl.BlockSpec(memory_space=pltpu.SMEM),
        ],
        out_specs=[
            pl.BlockSpec((N, D), lambda i: (0, 0)),
            pl.BlockSpec((N, 1), lambda i: (0, 0)),
        ],
        out_shape=[
            jax.ShapeDtypeStruct((N, D), jnp.float32),
            jax.ShapeDtypeStruct((N, 1), jnp.float32),
        ],
    )(yg2, Wv, bv.reshape(1, 1))


def _directional_scan(dr, mat_ref, start_ref, out_re_ref, out_im_ref):
    one = jnp.ones((NCH, 1), jnp.float32)
    zero = jnp.zeros((NCH, 1), jnp.float32)
    m00r, m00i = one, zero
    m01r, m01i = zero, zero
    m10r, m10i = zero, zero
    m11r, m11i = one, zero
    for j in range(CH):
        dj = dr[:, j:j + 1]
        n10r = dj * m10r - m10i - m00r
        n10i = dj * m10i + m10r - m00i
        n11r = dj * m11r - m11i - m01r
        n11i = dj * m11i + m11r - m01i
        m00r, m00i = m10r, m10i
        m01r, m01i = m11r, m11i
        m10r, m10i = n10r, n10i
        m11r, m11i = n11r, n11i
    mat_ref[:, 0:1] = m00r
    mat_ref[:, 1:2] = m00i
    mat_ref[:, 2:3] = m01r
    mat_ref[:, 3:4] = m01i
    mat_ref[:, 4:5] = m10r
    mat_ref[:, 5:6] = m10i
    mat_ref[:, 6:7] = m11r
    mat_ref[:, 7:8] = m11i

    def boundary(c, carry):
        are, aim = carry
        start_ref[pl.ds(c, 1), :] = jnp.concatenate([are, aim], axis=1)
        row = mat_ref[pl.ds(c, 1), :]
        numr = row[:, 0:1] * are - row[:, 1:2] * aim + row[:, 2:3]
        numi = row[:, 0:1] * aim + row[:, 1:2] * are + row[:, 3:4]
        denr = row[:, 4:5] * are - row[:, 5:6] * aim + row[:, 6:7]
        deni = row[:, 4:5] * aim + row[:, 5:6] * are + row[:, 7:8]
        nrm = denr * denr + deni * deni
        return ((numr * denr + numi * deni) / nrm,
                (numi * denr - numr * deni) / nrm)

    z11 = jnp.zeros((1, 1), jnp.float32)
    jax.lax.fori_loop(0, NCH, boundary, (z11, z11))

    are = start_ref[:, 0:1]
    aim = start_ref[:, 1:2]
    for j in range(CH):
        out_re_ref[:, j:j + 1] = are
        out_im_ref[:, j:j + 1] = aim
        x = dr[:, j:j + 1] - are
        y = 1.0 - aim
        nrm = x * x + y * y
        are = x / nrm
        aim = -y / nrm


def _scan_body(v_ref, vrev_ref, are_ref, aim_ref, bre_ref, bim_ref,
               mat_ref, start_ref):
    _directional_scan(2.0 - v_ref[...], mat_ref, start_ref, are_ref, aim_ref)
    _directional_scan(2.0 - vrev_ref[...], mat_ref, start_ref,
                      bre_ref, bim_ref)


def _bk_scan(v16, v16rev):
    return pl.pallas_call(
        _scan_body,
        grid=(1,),
        in_specs=[
            pl.BlockSpec((NCH, CH), lambda i: (0, 0)),
            pl.BlockSpec((NCH, CH), lambda i: (0, 0)),
        ],
        out_specs=[pl.BlockSpec((NCH, CH), lambda i: (0, 0))] * 4,
        out_shape=[jax.ShapeDtypeStruct((NCH, CH), jnp.float32)] * 4,
        scratch_shapes=[
            pltpu.VMEM((NCH, 8), jnp.float32),
            pltpu.VMEM((NCH, 2), jnp.float32),
        ],
    )(v16, v16rev)


def _combine_body(ffn_ref, v_ref, are_ref, aim_ref, rre_ref, rim_ref,
                  wo_ref, bo_ref, sc_ref, out_ref):
    x = (2.0 - v_ref[...]) - are_ref[...] - rre_ref[...]
    y = 1.0 - aim_ref[...] - rim_ref[...]
    nrm = x * x + y * y
    gre = jnp.clip(x / nrm, -FEAT_CLAMP, FEAT_CLAMP)
    gim = jnp.clip(-y / nrm, -FEAT_CLAMP, FEAT_CLAMP)
    spec = gre * wo_ref[0:1, :] + gim * wo_ref[1:2, :] + bo_ref[...]
    out_ref[...] = ffn_ref[...] + sc_ref[0, 0] * spec


def _combine(ffn, v2, are, aim, rre, rim, Wo, bo, bk_scale):
    return pl.pallas_call(
        _combine_body,
        grid=(1,),
        in_specs=[
            pl.BlockSpec((N, D), lambda i: (0, 0)),
            pl.BlockSpec((N, 1), lambda i: (0, 0)),
            pl.BlockSpec((N, 1), lambda i: (0, 0)),
            pl.BlockSpec((N, 1), lambda i: (0, 0)),
            pl.BlockSpec((N, 1), lambda i: (0, 0)),
            pl.BlockSpec((N, 1), lambda i: (0, 0)),
            pl.BlockSpec((2, D), lambda i: (0, 0)),
            pl.BlockSpec((1, D), lambda i: (0, 0)),
            pl.BlockSpec(memory_space=pltpu.SMEM),
        ],
        out_specs=pl.BlockSpec((N, D), lambda i: (0, 0)),
        out_shape=jax.ShapeDtypeStruct((N, D), jnp.float32),
    )(ffn, v2, are, aim, rre, rim, Wo, bo.reshape(1, D),
      bk_scale.reshape(1, 1))


def _dispatch_meta(i12, g12):
    eflat = i12.reshape(NK)
    gflat = g12.reshape(NK)
    oh = (eflat[:, None] == jnp.arange(E, dtype=jnp.int32)[None, :])
    oh = oh.astype(jnp.int32)
    csum = jnp.cumsum(oh, axis=0)
    rank = jnp.sum(oh * (csum - 1), axis=1)
    counts = csum[-1]
    padded = ((counts + BLK - 1) // BLK) * BLK
    pstart = jnp.concatenate(
        [jnp.zeros((1,), jnp.int32), jnp.cumsum(padded)])[:E]
    pos = pstart[eflat] + rank
    tok = jnp.arange(NK, dtype=jnp.int32) // K
    packed = jnp.stack([tok.astype(jnp.float32), gflat], axis=1)
    dst = jnp.zeros((M, 2), jnp.float32).at[pos].set(
        packed, unique_indices=True, mode="drop")
    tok_ids = dst[:, 0].astype(jnp.int32)
    gates_m = dst[:, 1]
    bstart = pstart // BLK
    be = jnp.sum(jnp.arange(NB, dtype=jnp.int32)[:, None] >= bstart[None, :],
                 axis=1).astype(jnp.int32) - 1
    pos_t = pos.reshape(N, K).T.reshape(NK)
    return tok_ids, gates_m, be, pos_t


def kernel(x, Wr, br, W1, b1, W2, b2, Wv, bv, Wo, bo, bk_scale):
    x2 = x.reshape(N, D)
    i12, g12 = _router(x2, Wr, br)
    tok_ids, gates_m, be, pos_t = _dispatch_meta(i12, g12)
    xg = _sc_gather(x2, tok_ids)
    ys = _grouped_ffn(be, xg, gates_m.reshape(M, 1), W1, b1, W2, b2)
    yg = _sc_gather(ys, pos_t)
    ffn, v2 = _sumv(yg.reshape(2, N, D), Wv, bv)
    v16 = v2.reshape(NCH, CH)
    v16rev = v16[::-1, ::-1]
    are, aim, bre, bim = _bk_scan(v16, v16rev)
    rre = bre[::-1, ::-1].reshape(N, 1)
    rim = bim[::-1, ::-1].reshape(N, 1)
    out = _combine(ffn, v2, are.reshape(N, 1), aim.reshape(N, 1),
                   rre, rim, Wo, bo, bk_scale)
    return out.reshape(x.shape)


# speedup vs baseline: 1.6577x; 1.6508x over previous
import jax
import jax.numpy as jnp
from jax.experimental import pallas as pl
from jax.experimental.pallas import tpu as pltpu

N = 2048
D = 768
E = 8
DFF = 3072
FB = 1024
NFB = DFF // FB
CH = 16
NCH = N // CH
V_MAX = 3.0
FEAT_CLAMP = 10.0
NEG_BIG = -1e30


def _ffn_body(x_ref, wr_ref, br_ref, w1_ref, b1_ref, w2_ref, b2_ref,
              wv_ref, bv_ref, out_ref, v_ref, wcol_ref, wall_ref):
    e = pl.program_id(0)
    fb = pl.program_id(1)

    @pl.when((e == 0) & (fb == 0))
    def _router():
        x = x_ref[...]
        logits = jnp.dot(x, wr_ref[...], preferred_element_type=jnp.float32)
        logits = logits + br_ref[...]
        idx = jax.lax.broadcasted_iota(jnp.int32, (N, E), 1)
        m1 = jnp.max(logits, axis=1, keepdims=True)
        i1 = jnp.min(jnp.where(logits == m1, idx, E), axis=1, keepdims=True)
        l2 = jnp.where(idx == i1, NEG_BIG, logits)
        m2 = jnp.max(l2, axis=1, keepdims=True)
        i2 = jnp.min(jnp.where(l2 == m2, idx, E), axis=1, keepdims=True)
        e2 = jnp.exp(m2 - m1)
        denom = 1.0 + e2
        g1 = 1.0 / denom
        g2 = e2 / denom
        wall_ref[...] = (jnp.where(idx == i1, g1, 0.0)
                         + jnp.where(idx == i2, g2, 0.0))
        out_ref[...] = jnp.zeros_like(out_ref)

    @pl.when(fb == 0)
    def _wcol():
        idx = jax.lax.broadcasted_iota(jnp.int32, (N, E), 1)
        sel = jnp.where(idx == e, wall_ref[...], 0.0)
        wcol_ref[...] = jnp.sum(sel, axis=1, keepdims=True)

    w = wcol_ref[...]
    h = jnp.dot(x_ref[...], w1_ref[0], preferred_element_type=jnp.float32)
    h = jnp.maximum(h + b1_ref[0], 0.0) * w
    out_ref[...] += jnp.dot(h, w2_ref[0], preferred_element_type=jnp.float32)

    @pl.when(fb == 0)
    def _bias2():
        out_ref[...] += w * b2_ref[0]

    @pl.when((e == E - 1) & (fb == NFB - 1))
    def _value_head():
        v = jnp.dot(out_ref[...], wv_ref[...],
                    preferred_element_type=jnp.float32)
        v_ref[...] = jnp.clip(v + bv_ref[0, 0], -V_MAX, V_MAX)


def _moe_ffn(x2, Wr, br, W1, b1, W2, b2, Wv, bv):
    return pl.pallas_call(
        _ffn_body,
        grid=(E, NFB),
        in_specs=[
            pl.BlockSpec((N, D), lambda e, f: (0, 0)),
            pl.BlockSpec((D, E), lambda e, f: (0, 0)),
            pl.BlockSpec((1, E), lambda e, f: (0, 0)),
            pl.BlockSpec((1, D, FB), lambda e, f: (e, 0, f)),
            pl.BlockSpec((1, 1, FB), lambda e, f: (e, 0, f)),
            pl.BlockSpec((1, FB, D), lambda e, f: (e, f, 0)),
            pl.BlockSpec((1, 1, D), lambda e, f: (e, 0, 0)),
            pl.BlockSpec((D, 1), lambda e, f: (0, 0)),
            pl.BlockSpec(memory_space=pltpu.SMEM),
        ],
        out_specs=[
            pl.BlockSpec((N, D), lambda e, f: (0, 0)),
            pl.BlockSpec((N, 1), lambda e, f: (0, 0)),
        ],
        out_shape=[
            jax.ShapeDtypeStruct((N, D), jnp.float32),
            jax.ShapeDtypeStruct((N, 1), jnp.float32),
        ],
        scratch_shapes=[
            pltpu.VMEM((N, 1), jnp.float32),
            pltpu.VMEM((N, E), jnp.float32),
        ],
    )(x2, Wr, br.reshape(1, E), W1, b1.reshape(E, 1, DFF), W2,
      b2.reshape(E, 1, D), Wv, bv.reshape(1, 1))


def _directional_scan(dr, mat_ref, start_ref, out_re_ref, out_im_ref):
    one = jnp.ones((NCH, 1), jnp.float32)
    zero = jnp.zeros((NCH, 1), jnp.float32)
    m00r, m00i = one, zero
    m01r, m01i = zero, zero
    m10r, m10i = zero, zero
    m11r, m11i = one, zero
    for j in range(CH):
        dj = dr[:, j:j + 1]
        n10r = dj * m10r - m10i - m00r
        n10i = dj * m10i + m10r - m00i
        n11r = dj * m11r - m11i - m01r
        n11i = dj * m11i + m11r - m01i
        m00r, m00i = m10r, m10i
        m01r, m01i = m11r, m11i
        m10r, m10i = n10r, n10i
        m11r, m11i = n11r, n11i
    mat_ref[:, 0:1] = m00r
    mat_ref[:, 1:2] = m00i
    mat_ref[:, 2:3] = m01r
    mat_ref[:, 3:4] = m01i
    mat_ref[:, 4:5] = m10r
    mat_ref[:, 5:6] = m10i
    mat_ref[:, 6:7] = m11r
    mat_ref[:, 7:8] = m11i

    def boundary(c, carry):
        are, aim = carry
        start_ref[pl.ds(c, 1), :] = jnp.concatenate([are, aim], axis=1)
        row = mat_ref[pl.ds(c, 1), :]
        numr = row[:, 0:1] * are - row[:, 1:2] * aim + row[:, 2:3]
        numi = row[:, 0:1] * aim + row[:, 1:2] * are + row[:, 3:4]
        denr = row[:, 4:5] * are - row[:, 5:6] * aim + row[:, 6:7]
        deni = row[:, 4:5] * aim + row[:, 5:6] * are + row[:, 7:8]
        nrm = denr * denr + deni * deni
        return ((numr * denr + numi * deni) / nrm,
                (numi * denr - numr * deni) / nrm)

    z11 = jnp.zeros((1, 1), jnp.float32)
    jax.lax.fori_loop(0, NCH, boundary, (z11, z11))

    are = start_ref[:, 0:1]
    aim = start_ref[:, 1:2]
    for j in range(CH):
        out_re_ref[:, j:j + 1] = are
        out_im_ref[:, j:j + 1] = aim
        x = dr[:, j:j + 1] - are
        y = 1.0 - aim
        nrm = x * x + y * y
        are = x / nrm
        aim = -y / nrm


def _scan_body(v_ref, vrev_ref, are_ref, aim_ref, bre_ref, bim_ref,
               mat_ref, start_ref):
    _directional_scan(2.0 - v_ref[...], mat_ref, start_ref, are_ref, aim_ref)
    _directional_scan(2.0 - vrev_ref[...], mat_ref, start_ref,
                      bre_ref, bim_ref)


def _bk_scan(v16, v16rev):
    return pl.pallas_call(
        _scan_body,
        grid=(1,),
        in_specs=[
            pl.BlockSpec((NCH, CH), lambda i: (0, 0)),
            pl.BlockSpec((NCH, CH), lambda i: (0, 0)),
        ],
        out_specs=[pl.BlockSpec((NCH, CH), lambda i: (0, 0))] * 4,
        out_shape=[jax.ShapeDtypeStruct((NCH, CH), jnp.float32)] * 4,
        scratch_shapes=[
            pltpu.VMEM((NCH, 8), jnp.float32),
            pltpu.VMEM((NCH, 2), jnp.float32),
        ],
    )(v16, v16rev)


def _combine_body(ffn_ref, v_ref, are_ref, aim_ref, rre_ref, rim_ref,
                  wo_ref, bo_ref, sc_ref, out_ref):
    x = (2.0 - v_ref[...]) - are_ref[...] - rre_ref[...]
    y = 1.0 - aim_ref[...] - rim_ref[...]
    nrm = x * x + y * y
    gre = jnp.clip(x / nrm, -FEAT_CLAMP, FEAT_CLAMP)
    gim = jnp.clip(-y / nrm, -FEAT_CLAMP, FEAT_CLAMP)
    spec = gre * wo_ref[0:1, :] + gim * wo_ref[1:2, :] + bo_ref[...]
    out_ref[...] = ffn_ref[...] + sc_ref[0, 0] * spec


def _combine(ffn, v2, are, aim, rre, rim, Wo, bo, bk_scale):
    return pl.pallas_call(
        _combine_body,
        grid=(1,),
        in_specs=[
            pl.BlockSpec((N, D), lambda i: (0, 0)),
            pl.BlockSpec((N, 1), lambda i: (0, 0)),
            pl.BlockSpec((N, 1), lambda i: (0, 0)),
            pl.BlockSpec((N, 1), lambda i: (0, 0)),
            pl.BlockSpec((N, 1), lambda i: (0, 0)),
            pl.BlockSpec((N, 1), lambda i: (0, 0)),
            pl.BlockSpec((2, D), lambda i: (0, 0)),
            pl.BlockSpec((1, D), lambda i: (0, 0)),
            pl.BlockSpec(memory_space=pltpu.SMEM),
        ],
        out_specs=pl.BlockSpec((N, D), lambda i: (0, 0)),
        out_shape=jax.ShapeDtypeStruct((N, D), jnp.float32),
    )(ffn, v2, are, aim, rre, rim, Wo, bo.reshape(1, D),
      bk_scale.reshape(1, 1))


def kernel(x, Wr, br, W1, b1, W2, b2, Wv, bv, Wo, bo, bk_scale):
    x2 = x.reshape(N, D)
    ffn, v2 = _moe_ffn(x2, Wr, br, W1, b1, W2, b2, Wv, bv)
    v16 = v2.reshape(NCH, CH)
    v16rev = v16[::-1, ::-1]
    are, aim, bre, bim = _bk_scan(v16, v16rev)
    rre = bre[::-1, ::-1].reshape(N, 1)
    rim = bim[::-1, ::-1].reshape(N, 1)
    out = _combine(ffn, v2, are.reshape(N, 1), aim.reshape(N, 1),
                   rre, rim, Wo, bo, bk_scale)
    return out.reshape(x.shape)


# speedup vs baseline: 1.6637x; 1.0036x over previous
import jax
import jax.numpy as jnp
from jax.experimental import pallas as pl
from jax.experimental.pallas import tpu as pltpu

N = 2048
D = 768
E = 8
DFF = 3072
FB = 1536
NFB = DFF // FB
CH = 16
NCH = N // CH
V_MAX = 3.0
FEAT_CLAMP = 10.0
NEG_BIG = -1e30


def _ffn_body(x_ref, wr_ref, br_ref, w1_ref, b1_ref, w2_ref, b2_ref,
              wv_ref, bv_ref, out_ref, v_ref, wcol_ref, wall_ref):
    e = pl.program_id(0)
    fb = pl.program_id(1)

    @pl.when((e == 0) & (fb == 0))
    def _router():
        x = x_ref[...]
        logits = jnp.dot(x, wr_ref[...], preferred_element_type=jnp.float32)
        logits = logits + br_ref[...]
        idx = jax.lax.broadcasted_iota(jnp.int32, (N, E), 1)
        m1 = jnp.max(logits, axis=1, keepdims=True)
        i1 = jnp.min(jnp.where(logits == m1, idx, E), axis=1, keepdims=True)
        l2 = jnp.where(idx == i1, NEG_BIG, logits)
        m2 = jnp.max(l2, axis=1, keepdims=True)
        i2 = jnp.min(jnp.where(l2 == m2, idx, E), axis=1, keepdims=True)
        e2 = jnp.exp(m2 - m1)
        denom = 1.0 + e2
        g1 = 1.0 / denom
        g2 = e2 / denom
        wall_ref[...] = (jnp.where(idx == i1, g1, 0.0)
                         + jnp.where(idx == i2, g2, 0.0))
        out_ref[...] = jnp.zeros_like(out_ref)

    @pl.when(fb == 0)
    def _wcol():
        idx = jax.lax.broadcasted_iota(jnp.int32, (N, E), 1)
        sel = jnp.where(idx == e, wall_ref[...], 0.0)
        wcol_ref[...] = jnp.sum(sel, axis=1, keepdims=True)

    w = wcol_ref[...]
    h = jnp.dot(x_ref[...], w1_ref[0], preferred_element_type=jnp.float32)
    h = jnp.maximum(h + b1_ref[0], 0.0) * w
    out_ref[...] += jnp.dot(h, w2_ref[0], preferred_element_type=jnp.float32)

    @pl.when(fb == 0)
    def _bias2():
        out_ref[...] += w * b2_ref[0]

    @pl.when((e == E - 1) & (fb == NFB - 1))
    def _value_head():
        v = jnp.dot(out_ref[...], wv_ref[...],
                    preferred_element_type=jnp.float32)
        v_ref[...] = jnp.clip(v + bv_ref[0, 0], -V_MAX, V_MAX)


def _moe_ffn(x2, Wr, br, W1, b1, W2, b2, Wv, bv):
    return pl.pallas_call(
        _ffn_body,
        grid=(E, NFB),
        in_specs=[
            pl.BlockSpec((N, D), lambda e, f: (0, 0)),
            pl.BlockSpec((D, E), lambda e, f: (0, 0)),
            pl.BlockSpec((1, E), lambda e, f: (0, 0)),
            pl.BlockSpec((1, D, FB), lambda e, f: (e, 0, f)),
            pl.BlockSpec((1, 1, FB), lambda e, f: (e, 0, f)),
            pl.BlockSpec((1, FB, D), lambda e, f: (e, f, 0)),
            pl.BlockSpec((1, 1, D), lambda e, f: (e, 0, 0)),
            pl.BlockSpec((D, 1), lambda e, f: (0, 0)),
            pl.BlockSpec(memory_space=pltpu.SMEM),
        ],
        out_specs=[
            pl.BlockSpec((N, D), lambda e, f: (0, 0)),
            pl.BlockSpec((N, 1), lambda e, f: (0, 0)),
        ],
        out_shape=[
            jax.ShapeDtypeStruct((N, D), jnp.float32),
            jax.ShapeDtypeStruct((N, 1), jnp.float32),
        ],
        scratch_shapes=[
            pltpu.VMEM((N, 1), jnp.float32),
            pltpu.VMEM((N, E), jnp.float32),
        ],
    )(x2, Wr, br.reshape(1, E), W1, b1.reshape(E, 1, DFF), W2,
      b2.reshape(E, 1, D), Wv, bv.reshape(1, 1))


def _directional_scan(dr, mat_ref, start_ref, out_re_ref, out_im_ref):
    one = jnp.ones((NCH, 1), jnp.float32)
    zero = jnp.zeros((NCH, 1), jnp.float32)
    m00r, m00i = one, zero
    m01r, m01i = zero, zero
    m10r, m10i = zero, zero
    m11r, m11i = one, zero
    for j in range(CH):
        dj = dr[:, j:j + 1]
        n10r = dj * m10r - m10i - m00r
        n10i = dj * m10i + m10r - m00i
        n11r = dj * m11r - m11i - m01r
        n11i = dj * m11i + m11r - m01i
        m00r, m00i = m10r, m10i
        m01r, m01i = m11r, m11i
        m10r, m10i = n10r, n10i
        m11r, m11i = n11r, n11i
    mat_ref[:, 0:1] = m00r
    mat_ref[:, 1:2] = m00i
    mat_ref[:, 2:3] = m01r
    mat_ref[:, 3:4] = m01i
    mat_ref[:, 4:5] = m10r
    mat_ref[:, 5:6] = m10i
    mat_ref[:, 6:7] = m11r
    mat_ref[:, 7:8] = m11i

    def boundary(c, carry):
        are, aim = carry
        start_ref[pl.ds(c, 1), :] = jnp.concatenate([are, aim], axis=1)
        row = mat_ref[pl.ds(c, 1), :]
        numr = row[:, 0:1] * are - row[:, 1:2] * aim + row[:, 2:3]
        numi = row[:, 0:1] * aim + row[:, 1:2] * are + row[:, 3:4]
        denr = row[:, 4:5] * are - row[:, 5:6] * aim + row[:, 6:7]
        deni = row[:, 4:5] * aim + row[:, 5:6] * are + row[:, 7:8]
        nrm = denr * denr + deni * deni
        return ((numr * denr + numi * deni) / nrm,
                (numi * denr - numr * deni) / nrm)

    z11 = jnp.zeros((1, 1), jnp.float32)
    jax.lax.fori_loop(0, NCH, boundary, (z11, z11))

    are = start_ref[:, 0:1]
    aim = start_ref[:, 1:2]
    for j in range(CH):
        out_re_ref[:, j:j + 1] = are
        out_im_ref[:, j:j + 1] = aim
        x = dr[:, j:j + 1] - are
        y = 1.0 - aim
        nrm = x * x + y * y
        are = x / nrm
        aim = -y / nrm


def _scan_body(v_ref, vrev_ref, are_ref, aim_ref, bre_ref, bim_ref,
               mat_ref, start_ref):
    _directional_scan(2.0 - v_ref[...], mat_ref, start_ref, are_ref, aim_ref)
    _directional_scan(2.0 - vrev_ref[...], mat_ref, start_ref,
                      bre_ref, bim_ref)


def _bk_scan(v16, v16rev):
    return pl.pallas_call(
        _scan_body,
        grid=(1,),
        in_specs=[
            pl.BlockSpec((NCH, CH), lambda i: (0, 0)),
            pl.BlockSpec((NCH, CH), lambda i: (0, 0)),
        ],
        out_specs=[pl.BlockSpec((NCH, CH), lambda i: (0, 0))] * 4,
        out_shape=[jax.ShapeDtypeStruct((NCH, CH), jnp.float32)] * 4,
        scratch_shapes=[
            pltpu.VMEM((NCH, 8), jnp.float32),
            pltpu.VMEM((NCH, 2), jnp.float32),
        ],
    )(v16, v16rev)


def _combine_body(ffn_ref, v_ref, are_ref, aim_ref, rre_ref, rim_ref,
                  wo_ref, bo_ref, sc_ref, out_ref):
    x = (2.0 - v_ref[...]) - are_ref[...] - rre_ref[...]
    y = 1.0 - aim_ref[...] - rim_ref[...]
    nrm = x * x + y * y
    gre = jnp.clip(x / nrm, -FEAT_CLAMP, FEAT_CLAMP)
    gim = jnp.clip(-y / nrm, -FEAT_CLAMP, FEAT_CLAMP)
    spec = gre * wo_ref[0:1, :] + gim * wo_ref[1:2, :] + bo_ref[...]
    out_ref[...] = ffn_ref[...] + sc_ref[0, 0] * spec


def _combine(ffn, v2, are, aim, rre, rim, Wo, bo, bk_scale):
    return pl.pallas_call(
        _combine_body,
        grid=(1,),
        in_specs=[
            pl.BlockSpec((N, D), lambda i: (0, 0)),
            pl.BlockSpec((N, 1), lambda i: (0, 0)),
            pl.BlockSpec((N, 1), lambda i: (0, 0)),
            pl.BlockSpec((N, 1), lambda i: (0, 0)),
            pl.BlockSpec((N, 1), lambda i: (0, 0)),
            pl.BlockSpec((N, 1), lambda i: (0, 0)),
            pl.BlockSpec((2, D), lambda i: (0, 0)),
            pl.BlockSpec((1, D), lambda i: (0, 0)),
            pl.BlockSpec(memory_space=pltpu.SMEM),
        ],
        out_specs=pl.BlockSpec((N, D), lambda i: (0, 0)),
        out_shape=jax.ShapeDtypeStruct((N, D), jnp.float32),
    )(ffn, v2, are, aim, rre, rim, Wo, bo.reshape(1, D),
      bk_scale.reshape(1, 1))


def kernel(x, Wr, br, W1, b1, W2, b2, Wv, bv, Wo, bo, bk_scale):
    x2 = x.reshape(N, D)
    ffn, v2 = _moe_ffn(x2, Wr, br, W1, b1, W2, b2, Wv, bv)
    v16 = v2.reshape(NCH, CH)
    v16rev = v16[::-1, ::-1]
    are, aim, bre, bim = _bk_scan(v16, v16rev)
    rre = bre[::-1, ::-1].reshape(N, 1)
    rim = bim[::-1, ::-1].reshape(N, 1)
    out = _combine(ffn, v2, are.reshape(N, 1), aim.reshape(N, 1),
                   rre, rim, Wo, bo, bk_scale)
    return out.reshape(x.shape)


# speedup vs baseline: 1.6659x; 1.0013x over previous
import jax
import jax.numpy as jnp
from jax.experimental import pallas as pl
from jax.experimental.pallas import tpu as pltpu

N = 2048
D = 768
E = 8
DFF = 3072
FB = 1536
NFB = DFF // FB
CH = 16
NCH = N // CH
V_MAX = 3.0
FEAT_CLAMP = 10.0
NEG_BIG = -1e30


def _ffn_body(x_ref, wr_ref, br_ref, w1_ref, b1_ref, w2_ref, b2_ref,
              wv_ref, bv_ref, out_ref, v_ref, wcol_ref, wall_ref):
    e = pl.program_id(0)
    fb = pl.program_id(1)

    @pl.when((e == 0) & (fb == 0))
    def _router():
        x = x_ref[...]
        logits = jnp.dot(x, wr_ref[...], preferred_element_type=jnp.float32)
        logits = logits + br_ref[...]
        idx = jax.lax.broadcasted_iota(jnp.int32, (N, E), 1)
        m1 = jnp.max(logits, axis=1, keepdims=True)
        i1 = jnp.min(jnp.where(logits == m1, idx, E), axis=1, keepdims=True)
        l2 = jnp.where(idx == i1, NEG_BIG, logits)
        m2 = jnp.max(l2, axis=1, keepdims=True)
        i2 = jnp.min(jnp.where(l2 == m2, idx, E), axis=1, keepdims=True)
        e2 = jnp.exp(m2 - m1)
        denom = 1.0 + e2
        g1 = 1.0 / denom
        g2 = e2 / denom
        wall_ref[...] = (jnp.where(idx == i1, g1, 0.0)
                         + jnp.where(idx == i2, g2, 0.0))
        out_ref[...] = jnp.zeros_like(out_ref)

    @pl.when(fb == 0)
    def _wcol():
        idx = jax.lax.broadcasted_iota(jnp.int32, (N, E), 1)
        sel = jnp.where(idx == e, wall_ref[...], 0.0)
        wcol_ref[...] = jnp.sum(sel, axis=1, keepdims=True)

    w = wcol_ref[...]
    h = jnp.dot(x_ref[...], w1_ref[0], preferred_element_type=jnp.float32)
    h = jnp.maximum(h + b1_ref[0], 0.0)
    out_ref[...] += w * jnp.dot(h, w2_ref[0],
                                preferred_element_type=jnp.float32)

    @pl.when(fb == 0)
    def _bias2():
        out_ref[...] += w * b2_ref[0]

    @pl.when((e == E - 1) & (fb == NFB - 1))
    def _value_head():
        v = jnp.dot(out_ref[...], wv_ref[...],
                    preferred_element_type=jnp.float32)
        v_ref[...] = jnp.clip(v + bv_ref[0, 0], -V_MAX, V_MAX)


def _moe_ffn(x2, Wr, br, W1, b1, W2, b2, Wv, bv):
    return pl.pallas_call(
        _ffn_body,
        grid=(E, NFB),
        in_specs=[
            pl.BlockSpec((N, D), lambda e, f: (0, 0)),
            pl.BlockSpec((D, E), lambda e, f: (0, 0)),
            pl.BlockSpec((1, E), lambda e, f: (0, 0)),
            pl.BlockSpec((1, D, FB), lambda e, f: (e, 0, f)),
            pl.BlockSpec((1, 1, FB), lambda e, f: (e, 0, f)),
            pl.BlockSpec((1, FB, D), lambda e, f: (e, f, 0)),
            pl.BlockSpec((1, 1, D), lambda e, f: (e, 0, 0)),
            pl.BlockSpec((D, 1), lambda e, f: (0, 0)),
            pl.BlockSpec(memory_space=pltpu.SMEM),
        ],
        out_specs=[
            pl.BlockSpec((N, D), lambda e, f: (0, 0)),
            pl.BlockSpec((N, 1), lambda e, f: (0, 0)),
        ],
        out_shape=[
            jax.ShapeDtypeStruct((N, D), jnp.float32),
            jax.ShapeDtypeStruct((N, 1), jnp.float32),
        ],
        scratch_shapes=[
            pltpu.VMEM((N, 1), jnp.float32),
            pltpu.VMEM((N, E), jnp.float32),
        ],
    )(x2, Wr, br.reshape(1, E), W1, b1.reshape(E, 1, DFF), W2,
      b2.reshape(E, 1, D), Wv, bv.reshape(1, 1))


def _directional_scan(dr, mat_ref, start_ref, out_re_ref, out_im_ref):
    one = jnp.ones((NCH, 1), jnp.float32)
    zero = jnp.zeros((NCH, 1), jnp.float32)
    m00r, m00i = one, zero
    m01r, m01i = zero, zero
    m10r, m10i = zero, zero
    m11r, m11i = one, zero
    for j in range(CH):
        dj = dr[:, j:j + 1]
        n10r = dj * m10r - m10i - m00r
        n10i = dj * m10i + m10r - m00i
        n11r = dj * m11r - m11i - m01r
        n11i = dj * m11i + m11r - m01i
        m00r, m00i = m10r, m10i
        m01r, m01i = m11r, m11i
        m10r, m10i = n10r, n10i
        m11r, m11i = n11r, n11i
    mat_ref[:, 0:1] = m00r
    mat_ref[:, 1:2] = m00i
    mat_ref[:, 2:3] = m01r
    mat_ref[:, 3:4] = m01i
    mat_ref[:, 4:5] = m10r
    mat_ref[:, 5:6] = m10i
    mat_ref[:, 6:7] = m11r
    mat_ref[:, 7:8] = m11i

    def boundary(c, carry):
        are, aim = carry
        start_ref[pl.ds(c, 1), :] = jnp.concatenate([are, aim], axis=1)
        row = mat_ref[pl.ds(c, 1), :]
        numr = row[:, 0:1] * are - row[:, 1:2] * aim + row[:, 2:3]
        numi = row[:, 0:1] * aim + row[:, 1:2] * are + row[:, 3:4]
        denr = row[:, 4:5] * are - row[:, 5:6] * aim + row[:, 6:7]
        deni = row[:, 4:5] * aim + row[:, 5:6] * are + row[:, 7:8]
        nrm = denr * denr + deni * deni
        return ((numr * denr + numi * deni) / nrm,
                (numi * denr - numr * deni) / nrm)

    z11 = jnp.zeros((1, 1), jnp.float32)
    jax.lax.fori_loop(0, NCH, boundary, (z11, z11))

    are = start_ref[:, 0:1]
    aim = start_ref[:, 1:2]
    for j in range(CH):
        out_re_ref[:, j:j + 1] = are
        out_im_ref[:, j:j + 1] = aim
        x = dr[:, j:j + 1] - are
        y = 1.0 - aim
        nrm = x * x + y * y
        are = x / nrm
        aim = -y / nrm


def _scan_body(v_ref, vrev_ref, are_ref, aim_ref, bre_ref, bim_ref,
               mat_ref, start_ref):
    _directional_scan(2.0 - v_ref[...], mat_ref, start_ref, are_ref, aim_ref)
    _directional_scan(2.0 - vrev_ref[...], mat_ref, start_ref,
                      bre_ref, bim_ref)


def _bk_scan(v16, v16rev):
    return pl.pallas_call(
        _scan_body,
        grid=(1,),
        in_specs=[
            pl.BlockSpec((NCH, CH), lambda i: (0, 0)),
            pl.BlockSpec((NCH, CH), lambda i: (0, 0)),
        ],
        out_specs=[pl.BlockSpec((NCH, CH), lambda i: (0, 0))] * 4,
        out_shape=[jax.ShapeDtypeStruct((NCH, CH), jnp.float32)] * 4,
        scratch_shapes=[
            pltpu.VMEM((NCH, 8), jnp.float32),
            pltpu.VMEM((NCH, 2), jnp.float32),
        ],
    )(v16, v16rev)


def _combine_body(ffn_ref, v_ref, are_ref, aim_ref, rre_ref, rim_ref,
                  wo_ref, bo_ref, sc_ref, out_ref):
    x = (2.0 - v_ref[...]) - are_ref[...] - rre_ref[...]
    y = 1.0 - aim_ref[...] - rim_ref[...]
    nrm = x * x + y * y
    gre = jnp.clip(x / nrm, -FEAT_CLAMP, FEAT_CLAMP)
    gim = jnp.clip(-y / nrm, -FEAT_CLAMP, FEAT_CLAMP)
    spec = gre * wo_ref[0:1, :] + gim * wo_ref[1:2, :] + bo_ref[...]
    out_ref[...] = ffn_ref[...] + sc_ref[0, 0] * spec


def _combine(ffn, v2, are, aim, rre, rim, Wo, bo, bk_scale):
    return pl.pallas_call(
        _combine_body,
        grid=(1,),
        in_specs=[
            pl.BlockSpec((N, D), lambda i: (0, 0)),
            pl.BlockSpec((N, 1), lambda i: (0, 0)),
            pl.BlockSpec((N, 1), lambda i: (0, 0)),
            pl.BlockSpec((N, 1), lambda i: (0, 0)),
            pl.BlockSpec((N, 1), lambda i: (0, 0)),
            pl.BlockSpec((N, 1), lambda i: (0, 0)),
            pl.BlockSpec((2, D), lambda i: (0, 0)),
            pl.BlockSpec((1, D), lambda i: (0, 0)),
            pl.BlockSpec(memory_space=pltpu.SMEM),
        ],
        out_specs=pl.BlockSpec((N, D), lambda i: (0, 0)),
        out_shape=jax.ShapeDtypeStruct((N, D), jnp.float32),
    )(ffn, v2, are, aim, rre, rim, Wo, bo.reshape(1, D),
      bk_scale.reshape(1, 1))


def kernel(x, Wr, br, W1, b1, W2, b2, Wv, bv, Wo, bo, bk_scale):
    x2 = x.reshape(N, D)
    ffn, v2 = _moe_ffn(x2, Wr, br, W1, b1, W2, b2, Wv, bv)
    v16 = v2.reshape(NCH, CH)
    v16rev = v16[::-1, ::-1]
    are, aim, bre, bim = _bk_scan(v16, v16rev)
    rre = bre[::-1, ::-1].reshape(N, 1)
    rim = bim[::-1, ::-1].reshape(N, 1)
    out = _combine(ffn, v2, are.reshape(N, 1), aim.reshape(N, 1),
                   rre, rim, Wo, bo, bk_scale)
    return out.reshape(x.shape)
